# baseline (speedup 1.0000x reference)
import jax
import jax.numpy as jnp
from jax.experimental import pallas as pl
from jax.experimental.pallas import tpu as pltpu

_BLK = 4096


def _dma_kernel(w_hbm, o_hbm, buf, in_sem, out_sem):
    nblk = pl.num_programs(0)
    i = pl.program_id(0)
    slot = jax.lax.rem(i, 2)
    nxt = jax.lax.rem(i + 1, 2)
    bsz = o_hbm.shape[0]

    def in_copy(blk_idx, buf_slot):
        return pltpu.make_async_copy(
            w_hbm.at[pl.ds(blk_idx * _BLK, _BLK), :],
            buf.at[buf_slot],
            in_sem.at[buf_slot],
        )

    def out_copy(b, blk_idx, buf_slot):
        return pltpu.make_async_copy(
            buf.at[buf_slot],
            o_hbm.at[b, pl.ds(blk_idx * _BLK, _BLK), :],
            out_sem.at[buf_slot, b],
        )

    @pl.when(i == 0)
    def _():
        in_copy(0, 0).start()

    in_copy(i, slot).wait()

    for b in range(bsz):
        out_copy(b, i, slot).start()

    @pl.when(i + 1 < nblk)
    def _():
        @pl.when(i >= 1)
        def _():
            for b in range(bsz):
                out_copy(b, i - 1, nxt).wait()

        in_copy(i + 1, nxt).start()

    @pl.when(i + 1 == nblk)
    def _():
        @pl.when(i >= 1)
        def _():
            for b in range(bsz):
                out_copy(b, i - 1, nxt).wait()

        for b in range(bsz):
            out_copy(b, i, slot).wait()


def kernel(input, embedding_weight):
    bsz, seq_len = input.shape
    d = embedding_weight.shape[1]
    nblk = seq_len // _BLK
    return pl.pallas_call(
        _dma_kernel,
        grid=(nblk,),
        in_specs=[pl.BlockSpec(memory_space=pltpu.MemorySpace.HBM)],
        out_specs=pl.BlockSpec(memory_space=pltpu.MemorySpace.HBM),
        out_shape=jax.ShapeDtypeStruct((bsz, seq_len, d), embedding_weight.dtype),
        scratch_shapes=[
            pltpu.MemorySpace.VMEM((2, _BLK, d), embedding_weight.dtype),
            pltpu.SemaphoreType.DMA((2,)),
            pltpu.SemaphoreType.DMA((2, bsz)),
        ],
    )(embedding_weight[:seq_len])


# speedup vs baseline: 1.0156x; 1.0156x over previous
import jax
import jax.numpy as jnp
from jax.experimental import pallas as pl
from jax.experimental.pallas import tpu as pltpu

_CHUNKS = (256, 256, 512, 1024, 2048, 4096)


def _dma_kernel(w_hbm, o_hbm, buf, in_sem, out_sem):
    bsz = o_hbm.shape[0]
    starts = []
    s = 0
    for c in _CHUNKS:
        starts.append(s)
        s += c

    def in_copy(ci):
        s0, c = starts[ci], _CHUNKS[ci]
        return pltpu.make_async_copy(
            w_hbm.at[pl.ds(s0, c), :],
            buf.at[pl.ds(s0, c), :],
            in_sem.at[ci],
        )

    def out_copy(b, ci):
        s0, c = starts[ci], _CHUNKS[ci]
        return pltpu.make_async_copy(
            buf.at[pl.ds(s0, c), :],
            o_hbm.at[b, pl.ds(s0, c), :],
            out_sem.at[ci, b],
        )

    for ci in range(len(_CHUNKS)):
        in_copy(ci).start()
    for ci in range(len(_CHUNKS)):
        in_copy(ci).wait()
        for b in range(bsz):
            out_copy(b, ci).start()
    for ci in range(len(_CHUNKS)):
        for b in range(bsz):
            out_copy(b, ci).wait()


def kernel(input, embedding_weight):
    bsz, seq_len = input.shape
    d = embedding_weight.shape[1]
    return pl.pallas_call(
        _dma_kernel,
        grid=(1,),
        in_specs=[pl.BlockSpec(memory_space=pltpu.MemorySpace.HBM)],
        out_specs=pl.BlockSpec(memory_space=pltpu.MemorySpace.HBM),
        out_shape=jax.ShapeDtypeStruct((bsz, seq_len, d), embedding_weight.dtype),
        scratch_shapes=[
            pltpu.MemorySpace.VMEM((seq_len, d), embedding_weight.dtype),
            pltpu.SemaphoreType.DMA((len(_CHUNKS),)),
            pltpu.SemaphoreType.DMA((len(_CHUNKS), bsz)),
        ],
    )(embedding_weight[:seq_len])


# speedup vs baseline: 1.0167x; 1.0011x over previous
import jax
import jax.numpy as jnp
from jax.experimental import pallas as pl
from jax.experimental.pallas import tpu as pltpu

_CHUNKS = (128, 128, 256, 512, 1024, 2048, 4096)


def _dma_kernel(w_hbm, o_hbm, buf, in_sem, out_sem):
    bsz = o_hbm.shape[0]
    starts = []
    s = 0
    for c in _CHUNKS:
        starts.append(s)
        s += c

    def in_copy(ci):
        s0, c = starts[ci], _CHUNKS[ci]
        return pltpu.make_async_copy(
            w_hbm.at[pl.ds(s0, c), :],
            buf.at[pl.ds(s0, c), :],
            in_sem.at[ci],
        )

    def out_copy(b, ci):
        s0, c = starts[ci], _CHUNKS[ci]
        return pltpu.make_async_copy(
            buf.at[pl.ds(s0, c), :],
            o_hbm.at[b, pl.ds(s0, c), :],
            out_sem.at[ci, b],
        )

    for ci in range(len(_CHUNKS)):
        in_copy(ci).start()
    for ci in range(len(_CHUNKS)):
        in_copy(ci).wait()
        for b in range(bsz):
            out_copy(b, ci).start()
    for ci in range(len(_CHUNKS)):
        for b in range(bsz):
            out_copy(b, ci).wait()


def kernel(input, embedding_weight):
    bsz, seq_len = input.shape
    d = embedding_weight.shape[1]
    return pl.pallas_call(
        _dma_kernel,
        grid=(1,),
        in_specs=[pl.BlockSpec(memory_space=pltpu.MemorySpace.HBM)],
        out_specs=pl.BlockSpec(memory_space=pltpu.MemorySpace.HBM),
        out_shape=jax.ShapeDtypeStruct((bsz, seq_len, d), embedding_weight.dtype),
        scratch_shapes=[
            pltpu.MemorySpace.VMEM((seq_len, d), embedding_weight.dtype),
            pltpu.SemaphoreType.DMA((len(_CHUNKS),)),
            pltpu.SemaphoreType.DMA((len(_CHUNKS), bsz)),
        ],
    )(embedding_weight[:seq_len])


# speedup vs baseline: 1.0168x; 1.0002x over previous
import jax
import jax.numpy as jnp
from jax.experimental import pallas as pl
from jax.experimental.pallas import tpu as pltpu

_CHUNKS = (256, 256, 512, 1024, 2048, 4096)


def _dma_kernel(w_hbm, o_hbm, buf, in_sem, out_sem):
    bsz = o_hbm.shape[0]
    starts = []
    s = 0
    for c in _CHUNKS:
        starts.append(s)
        s += c

    def in_copy(ci):
        s0, c = starts[ci], _CHUNKS[ci]
        return pltpu.make_async_copy(
            w_hbm.at[pl.ds(s0, c), :],
            buf.at[pl.ds(s0, c), :],
            in_sem.at[ci],
        )

    def out_copy(b, ci):
        s0, c = starts[ci], _CHUNKS[ci]
        return pltpu.make_async_copy(
            buf.at[pl.ds(s0, c), :],
            o_hbm.at[b, pl.ds(s0, c), :],
            out_sem.at[ci, b],
        )

    for ci in range(len(_CHUNKS)):
        in_copy(ci).start()
    for ci in range(len(_CHUNKS)):
        in_copy(ci).wait()
        for b in range(bsz):
            out_copy(b, ci).start()
    for ci in range(len(_CHUNKS)):
        for b in range(bsz):
            out_copy(b, ci).wait()


def kernel(input, embedding_weight):
    bsz, seq_len = input.shape
    d = embedding_weight.shape[1]
    return pl.pallas_call(
        _dma_kernel,
        grid=(1,),
        in_specs=[pl.BlockSpec(memory_space=pltpu.MemorySpace.HBM)],
        out_specs=pl.BlockSpec(memory_space=pltpu.MemorySpace.HBM),
        out_shape=jax.ShapeDtypeStruct((bsz, seq_len, d), embedding_weight.dtype),
        scratch_shapes=[
            pltpu.MemorySpace.VMEM((seq_len, d), embedding_weight.dtype),
            pltpu.SemaphoreType.DMA((len(_CHUNKS),)),
            pltpu.SemaphoreType.DMA((len(_CHUNKS), bsz)),
        ],
    )(embedding_weight[:seq_len])


# speedup vs baseline: 1.0189x; 1.0020x over previous
import functools

import jax
import jax.numpy as jnp
from jax.experimental import pallas as pl
from jax.experimental.pallas import tpu as pltpu

def _ramp_chunks(seq_len):
    chunks = []
    rem, c = seq_len, 256
    while rem > 0:
        step = min(c, rem)
        chunks.append(step)
        rem -= step
        if len(chunks) >= 2:
            c = min(c * 2, 4096)
    return tuple(chunks)


def _dma_kernel(w_hbm, o_hbm, buf, in_sem, out_sem, *, chunks):
    bsz = o_hbm.shape[0]
    starts = []
    s = 0
    for c in chunks:
        starts.append(s)
        s += c

    def in_copy(ci):
        s0, c = starts[ci], chunks[ci]
        return pltpu.make_async_copy(
            w_hbm.at[pl.ds(s0, c), :],
            buf.at[pl.ds(s0, c), :],
            in_sem.at[ci],
        )

    def out_copy(b, ci):
        s0, c = starts[ci], chunks[ci]
        return pltpu.make_async_copy(
            buf.at[pl.ds(s0, c), :],
            o_hbm.at[b, pl.ds(s0, c), :],
            out_sem.at[ci, b],
        )

    for ci in range(len(chunks)):
        in_copy(ci).start()
    for ci in range(len(chunks)):
        in_copy(ci).wait()
        for b in range(bsz):
            out_copy(b, ci).start()
    for ci in range(len(chunks)):
        for b in range(bsz):
            out_copy(b, ci).wait()


def kernel(input, embedding_weight):
    bsz, seq_len = input.shape
    d = embedding_weight.shape[1]
    chunks = _ramp_chunks(seq_len)
    return pl.pallas_call(
        functools.partial(_dma_kernel, chunks=chunks),
        grid=(1,),
        in_specs=[pl.BlockSpec(memory_space=pltpu.MemorySpace.HBM)],
        out_specs=pl.BlockSpec(memory_space=pltpu.MemorySpace.HBM),
        out_shape=jax.ShapeDtypeStruct((bsz, seq_len, d), embedding_weight.dtype),
        scratch_shapes=[
            pltpu.MemorySpace.VMEM((seq_len, d), embedding_weight.dtype),
            pltpu.SemaphoreType.DMA((len(chunks),)),
            pltpu.SemaphoreType.DMA((len(chunks), bsz)),
        ],
    )(embedding_weight[:seq_len])
